# baseline (speedup 1.0000x reference)
import functools

import jax
import jax.numpy as jnp
from jax import lax
from jax.experimental import pallas as pl
from jax.experimental.pallas import tpu as pltpu
from jax.experimental.pallas import tpu_sc as plsc

NC = 2
NS = 16
NW = NC * NS


def _sc_body(B, N, C, K, dist_hbm, idxk_hbm, r_hbm, xt_hbm,
             med_hbm, xr_hbm,
             idxk_v, fidx_v, gpos_v, gidx_v, xidx_v, dg_v, r_v, med_v,
             xr_v, sem):
    BN = B * N
    PW = BN // NW
    wid = lax.axis_index("c") * NS + lax.axis_index("s")
    base = wid * PW
    b = base // N
    bN = b * N

    pltpu.sync_copy(idxk_hbm.at[pl.ds(base * K, PW * K)], idxk_v)
    pltpu.sync_copy(r_hbm.at[pl.ds(base, PW)], r_v)

    NCHUNK = (PW * K) // 128

    def mk_fidx(j, carry):
        for h in range(8):
            p = j * 4 + h // 2
            v = idxk_v[pl.ds(j * 128 + h * 16, 16)]
            fidx_v[j, pl.ds(h * 16, 16)] = v + (base + p) * N
        return carry
    lax.fori_loop(0, NCHUNK, mk_fidx, 0, unroll=False)

    def mk_gpos(t, carry):
        pvec = t * 16 + lax.iota(jnp.int32, 16)
        rvec = r_v[pl.ds(t * 16, 16)]
        gpos_v[t // 8, pl.ds((t % 8) * 16, 16)] = (base + pvec) * K + rvec
        return carry
    lax.fori_loop(0, PW // 16, mk_gpos, 0, unroll=False)

    GCHUNK = PW // 128
    for j in range(GCHUNK):
        pltpu.make_async_copy(idxk_hbm.at[gpos_v.at[j]],
                              gidx_v.at[pl.ds(j * 128, 128)], sem).start()
    for j in range(GCHUNK):
        pltpu.make_async_copy(idxk_hbm.at[gpos_v.at[j]],
                              gidx_v.at[pl.ds(j * 128, 128)], sem).wait()

    def mk_xidx(t, carry):
        q = gidx_v[pl.ds(t * 16, 16)] + bN
        for c in range(C):
            row = (c * PW) // 128 + t // 8
            col = (t % 8) * 16
            xidx_v[row, pl.ds(col, 16)] = q + c * BN
        return carry
    lax.fori_loop(0, PW // 16, mk_xidx, 0, unroll=False)

    XCHUNK = (C * PW) // 128

    def d_copy(j):
        return pltpu.make_async_copy(
            dist_hbm.at[fidx_v.at[j]], dg_v.at[pl.ds(j * 128, 128)], sem)

    def x_copy(j):
        return pltpu.make_async_copy(
            xt_hbm.at[xidx_v.at[j]], xr_v.at[pl.ds(j * 128, 128)], sem)

    def gather_chunk(cidx, carry):
        for u in range(8):
            d_copy(cidx * 8 + u).start()
        for u in range(8):
            d_copy(cidx * 8 + u).wait()
        return carry
    lax.fori_loop(0, NCHUNK // 8, gather_chunk, 0, unroll=False)

    def xgather_chunk(cidx, carry):
        for u in range(4):
            x_copy(cidx * 4 + u).start()
        for u in range(4):
            x_copy(cidx * 4 + u).wait()
        return carry
    lax.fori_loop(0, XCHUNK // 4, xgather_chunk, 0, unroll=False)

    lane = lax.iota(jnp.int32, 16)

    def median_grp(g, carry):
        medvec = jnp.zeros((16,), jnp.float32)
        for u in range(16):
            p = g * 16 + u
            a = dg_v[pl.ds(p * K, 16)]
            bb = dg_v[pl.ds(p * K + 16, 16)]
            lo = jnp.minimum(jnp.sort(a), jnp.flip(jnp.sort(bb)))
            medvec = jnp.where(lane == u, jnp.max(lo), medvec)
        med_v[pl.ds(g * 16, 16)] = medvec
        return carry
    lax.fori_loop(0, PW // 16, median_grp, 0, unroll=False)

    pltpu.sync_copy(med_v, med_hbm.at[pl.ds(base, PW)])
    for c in range(C):
        pltpu.sync_copy(xr_v.at[pl.ds(c * PW, PW)],
                        xr_hbm.at[pl.ds(c * BN + base, PW)])


def _sc_call(B, N, C, K, dist_flat, idxk_flat, r_flat, xt_flat):
    BN = B * N
    PW = BN // NW
    mesh = plsc.VectorSubcoreMesh(core_axis_name="c", subcore_axis_name="s")
    kern = pl.kernel(
        functools.partial(_sc_body, B, N, C, K),
        out_type=(
            jax.ShapeDtypeStruct((BN,), jnp.float32),
            jax.ShapeDtypeStruct((C * BN,), jnp.float32),
        ),
        mesh=mesh,
        compiler_params=pltpu.CompilerParams(needs_layout_passes=False),
        scratch_types=[
            pltpu.VMEM((PW * K,), jnp.int32),
            pltpu.VMEM((PW * K // 128, 128), jnp.int32),
            pltpu.VMEM((PW // 128, 128), jnp.int32),
            pltpu.VMEM((PW,), jnp.int32),
            pltpu.VMEM((C * PW // 128, 128), jnp.int32),
            pltpu.VMEM((PW * K,), jnp.float32),
            pltpu.VMEM((PW,), jnp.int32),
            pltpu.VMEM((PW,), jnp.float32),
            pltpu.VMEM((C * PW,), jnp.float32),
            pltpu.SemaphoreType.DMA,
        ],
    )
    return kern(dist_flat, idxk_flat, r_flat, xt_flat)


def _tc_body(xt_ref, nt_ref, xr_ref, med_ref, out_ref):
    xt = xt_ref[...]
    nt = nt_ref[...]
    xv = xr_ref[...] - xt
    dot = jnp.sum(xv * nt, axis=0, keepdims=True)
    xp = xv - dot * nt
    n2 = jnp.sum(xp * xp, axis=0, keepdims=True)
    norms = jnp.maximum(jnp.sqrt(n2), 1e-6)
    half = norms * 0.5
    mk = jnp.sqrt(med_ref[...])
    clamped = jnp.where(half > mk, mk, half)
    out_ref[...] = xt + xp * (clamped / norms)


def _tc_call(xt, nt, xr, med):
    C, BN = xt.shape
    return pl.pallas_call(
        _tc_body,
        out_shape=jax.ShapeDtypeStruct((C, BN), jnp.float32),
    )(xt, nt, xr, med)


def kernel(x, distance, idx_k, normals):
    B, N, C = x.shape
    K = idx_k.shape[-1]
    BN = B * N
    r = jax.random.randint(jax.random.key(42), (B, N, 1), 0, K,
                           dtype=jnp.int32)
    xt = x.transpose(2, 0, 1).reshape(C, BN)
    nt = normals.transpose(2, 0, 1).reshape(C, BN)
    med, xr = _sc_call(B, N, C, K,
                       distance.reshape(BN * N),
                       idx_k.reshape(BN * K),
                       r.reshape(BN),
                       xt.reshape(C * BN))
    out_t = _tc_call(xt, nt, xr.reshape(C, BN), med.reshape(1, BN))
    return out_t.reshape(C, B, N).transpose(1, 2, 0)


# speedup vs baseline: 1.5640x; 1.5640x over previous
import functools

import jax
import jax.numpy as jnp
from jax import lax
from jax.experimental import pallas as pl
from jax.experimental.pallas import tpu as pltpu
from jax.experimental.pallas import tpu_sc as plsc

NC = 2
NS = 16
NW = NC * NS


def _sc_body(B, N, C, K, dist_hbm, idxk_hbm, r_hbm, xt_hbm,
             med_hbm, xr_hbm,
             idxk_v, gpos_v, gidx_v, xidx_v, r_v, med_v, xr_v,
             slab0, slab1, sem_a, sem_b, sem_g, sem_x):
    BN = B * N
    PW = BN // NW
    wid = lax.axis_index("c") * NS + lax.axis_index("s")
    base = wid * PW
    b = base // N
    n0 = base - b * N
    bN = b * N

    pltpu.sync_copy(idxk_hbm.at[pl.ds(base * K, PW * K)], idxk_v)
    pltpu.sync_copy(r_hbm.at[pl.ds(base, PW)], r_v)

    def mk_gpos(t, carry):
        pvec = t * 16 + lax.iota(jnp.int32, 16)
        rvec = r_v[pl.ds(t * 16, 16)]
        gpos_v[t // 8, pl.ds((t % 8) * 16, 16)] = (base + pvec) * K + rvec
        return carry
    lax.fori_loop(0, PW // 16, mk_gpos, 0, unroll=False)

    GCHUNK = PW // 128
    for j in range(GCHUNK):
        pltpu.make_async_copy(idxk_hbm.at[gpos_v.at[j]],
                              gidx_v.at[pl.ds(j * 128, 128)], sem_g).start()
    for j in range(GCHUNK):
        pltpu.make_async_copy(idxk_hbm.at[gpos_v.at[j]],
                              gidx_v.at[pl.ds(j * 128, 128)], sem_g).wait()

    def mk_xidx(t, carry):
        q = gidx_v[pl.ds(t * 16, 16)] + bN
        for c in range(C):
            row = (c * PW) // 128 + t // 8
            col = (t % 8) * 16
            xidx_v[row, pl.ds(col, 16)] = q + c * BN
        return carry
    lax.fori_loop(0, PW // 16, mk_xidx, 0, unroll=False)

    XCHUNK = (C * PW) // 128
    def x_copy(j):
        return pltpu.make_async_copy(
            xt_hbm.at[xidx_v.at[j]], xr_v.at[pl.ds(j * 128, 128)], sem_x)
    for j in range(XCHUNK):
        x_copy(j).start()

    NSLAB = PW // 8
    slabs = (slab0, slab1)
    sems = (sem_a, sem_b)

    def slab_copy(g, slot):
        return pltpu.make_async_copy(
            dist_hbm.at[b, pl.ds(n0 + g * 8, 8)], slabs[slot], sems[slot])

    lane = lax.iota(jnp.int32, 16)

    def process(g, slot, lane0, medvec):
        sl = slabs[slot]
        for u in range(8):
            p = g * 8 + u
            iv0 = idxk_v[pl.ds(p * K, 16)]
            iv1 = idxk_v[pl.ds(p * K + 16, 16)]
            urow = jnp.full((16,), u, jnp.int32)
            a = plsc.load_gather(sl, [urow, iv0])
            bb = plsc.load_gather(sl, [urow, iv1])
            lo = jnp.minimum(jnp.sort(a), jnp.flip(jnp.sort(bb)))
            medvec = jnp.where(lane == lane0 + u, jnp.max(lo), medvec)
        return medvec

    slab_copy(0, 0).start()

    def pair(i, carry):
        g0 = 2 * i
        g1 = 2 * i + 1
        slab_copy(g1, 1).start()
        slab_copy(g0, 0).wait()
        mv = process(g0, 0, 0, jnp.zeros((16,), jnp.float32))

        @pl.when(g0 + 2 < NSLAB)
        def _():
            slab_copy(g0 + 2, 0).start()

        slab_copy(g1, 1).wait()
        mv = process(g1, 1, 8, mv)
        med_v[pl.ds(i * 16, 16)] = mv
        return carry
    lax.fori_loop(0, NSLAB // 2, pair, 0, unroll=False)

    for j in range(XCHUNK):
        x_copy(j).wait()
    pltpu.sync_copy(med_v, med_hbm.at[pl.ds(base, PW)])
    for c in range(C):
        pltpu.sync_copy(xr_v.at[pl.ds(c * PW, PW)],
                        xr_hbm.at[pl.ds(c * BN + base, PW)])


def _sc_call(B, N, C, K, distance, idxk_flat, r_flat, xt_flat):
    BN = B * N
    PW = BN // NW
    mesh = plsc.VectorSubcoreMesh(core_axis_name="c", subcore_axis_name="s")
    kern = pl.kernel(
        functools.partial(_sc_body, B, N, C, K),
        out_type=(
            jax.ShapeDtypeStruct((BN,), jnp.float32),
            jax.ShapeDtypeStruct((C * BN,), jnp.float32),
        ),
        mesh=mesh,
        compiler_params=pltpu.CompilerParams(needs_layout_passes=False),
        scratch_types=[
            pltpu.VMEM((PW * K,), jnp.int32),
            pltpu.VMEM((PW // 128, 128), jnp.int32),
            pltpu.VMEM((PW,), jnp.int32),
            pltpu.VMEM((C * PW // 128, 128), jnp.int32),
            pltpu.VMEM((PW,), jnp.int32),
            pltpu.VMEM((PW,), jnp.float32),
            pltpu.VMEM((C * PW,), jnp.float32),
            pltpu.VMEM((8, N), jnp.float32),
            pltpu.VMEM((8, N), jnp.float32),
            pltpu.SemaphoreType.DMA,
            pltpu.SemaphoreType.DMA,
            pltpu.SemaphoreType.DMA,
            pltpu.SemaphoreType.DMA,
        ],
    )
    return kern(distance, idxk_flat, r_flat, xt_flat)


def _tc_body(xt_ref, nt_ref, xr_ref, med_ref, out_ref):
    xt = xt_ref[...]
    nt = nt_ref[...]
    xv = xr_ref[...] - xt
    dot = jnp.sum(xv * nt, axis=0, keepdims=True)
    xp = xv - dot * nt
    n2 = jnp.sum(xp * xp, axis=0, keepdims=True)
    norms = jnp.maximum(jnp.sqrt(n2), 1e-6)
    half = norms * 0.5
    mk = jnp.sqrt(med_ref[...])
    clamped = jnp.where(half > mk, mk, half)
    out_ref[...] = xt + xp * (clamped / norms)


def _tc_call(xt, nt, xr, med):
    C, BN = xt.shape
    return pl.pallas_call(
        _tc_body,
        out_shape=jax.ShapeDtypeStruct((C, BN), jnp.float32),
    )(xt, nt, xr, med)


def kernel(x, distance, idx_k, normals):
    B, N, C = x.shape
    K = idx_k.shape[-1]
    BN = B * N
    r = jax.random.randint(jax.random.key(42), (B, N, 1), 0, K,
                           dtype=jnp.int32)
    xt = x.transpose(2, 0, 1).reshape(C, BN)
    nt = normals.transpose(2, 0, 1).reshape(C, BN)
    med, xr = _sc_call(B, N, C, K,
                       distance,
                       idx_k.reshape(BN * K),
                       r.reshape(BN),
                       xt.reshape(C * BN))
    out_t = _tc_call(xt, nt, xr.reshape(C, BN), med.reshape(1, BN))
    return out_t.reshape(C, B, N).transpose(1, 2, 0)
